# baseline (speedup 1.0000x reference)
import functools

import jax
import jax.numpy as jnp
from jax import lax
from jax.experimental import pallas as pl
from jax.experimental.pallas import tpu as pltpu
from jax.experimental.pallas import tpu_sc as plsc

N_FRAMES = 1000
N_RAYS = 16384
L = 16
NC = 1
NS = 16
NW = NC * NS
RAYS_PER_W = N_RAYS // NW
GROUPS = RAYS_PER_W // L
TABLE_WORDS = N_FRAMES * 7


def _sc_body(table_hbm, rays_hbm, idx_hbm, mask_hbm, out_hbm,
             table_v, rays_v, idx_v, mask_v, out_v,
             sem_t, sem_i, sem_m, sem_r, sem_o):
    wid = lax.axis_index("s") * NC + lax.axis_index("c")
    rbase = wid * RAYS_PER_W

    cp_t = pltpu.make_async_copy(table_hbm, table_v, sem_t)
    cp_i = pltpu.make_async_copy(idx_hbm.at[pl.ds(rbase, RAYS_PER_W)],
                                 idx_v, sem_i)
    cp_m = pltpu.make_async_copy(mask_hbm.at[pl.ds(rbase, RAYS_PER_W)],
                                 mask_v, sem_m)
    cp_r = [pltpu.make_async_copy(
        rays_hbm.at[pl.ds(c * N_RAYS + rbase, RAYS_PER_W)],
        rays_v.at[pl.ds(c * RAYS_PER_W, RAYS_PER_W)], sem_r)
        for c in range(6)]
    cp_t.start(); cp_i.start(); cp_m.start()
    for cp in cp_r:
        cp.start()
    cp_t.wait(); cp_i.wait(); cp_m.wait()
    for cp in cp_r:
        cp.wait()

    zero = jnp.zeros((L,), jnp.float32)
    one = jnp.ones((L,), jnp.float32)

    @plsc.parallel_loop(0, GROUPS, unroll=1)
    def body(g):
        s = g * L
        idx = idx_v[pl.ds(s, L)]
        m = mask_v[pl.ds(s, L)] == 1
        c = [plsc.load_gather(table_v, [idx + k * N_FRAMES])
             for k in range(7)]
        tx = jnp.where(m, c[0], zero)
        ty = jnp.where(m, c[1], zero)
        tz = jnp.where(m, c[2], zero)
        qx = jnp.where(m, c[3], zero)
        qy = jnp.where(m, c[4], zero)
        qz = jnp.where(m, c[5], zero)
        qw = jnp.where(m, c[6], one)

        r = [rays_v[pl.ds(k * RAYS_PER_W + s, L)] for k in range(6)]

        xx, yy, zz = qx * qx, qy * qy, qz * qz
        xy, xz, yz = qx * qy, qx * qz, qy * qz
        wx, wy, wz = qw * qx, qw * qy, qw * qz
        two = jnp.float32(2.0)
        r00 = 1 - two * (yy + zz); r01 = two * (xy - wz); r02 = two * (xz + wy)
        r10 = two * (xy + wz); r11 = 1 - two * (xx + zz); r12 = two * (yz - wx)
        r20 = two * (xz - wy); r21 = two * (yz + wx); r22 = 1 - two * (xx + yy)

        out_v[pl.ds(0 * RAYS_PER_W + s, L)] = r[0] + tx
        out_v[pl.ds(1 * RAYS_PER_W + s, L)] = r[1] + ty
        out_v[pl.ds(2 * RAYS_PER_W + s, L)] = r[2] + tz
        out_v[pl.ds(3 * RAYS_PER_W + s, L)] = (
            r00 * r[3] + r01 * r[4] + r02 * r[5])
        out_v[pl.ds(4 * RAYS_PER_W + s, L)] = (
            r10 * r[3] + r11 * r[4] + r12 * r[5])
        out_v[pl.ds(5 * RAYS_PER_W + s, L)] = (
            r20 * r[3] + r21 * r[4] + r22 * r[5])

    cp_o = [pltpu.make_async_copy(
        out_v.at[pl.ds(c * RAYS_PER_W, RAYS_PER_W)],
        out_hbm.at[pl.ds(c * N_RAYS + rbase, RAYS_PER_W)], sem_o)
        for c in range(6)]
    for cp in cp_o:
        cp.start()
    for cp in cp_o:
        cp.wait()


_sc_kernel = functools.partial(
    pl.kernel,
    out_type=jax.ShapeDtypeStruct((N_RAYS * 6,), jnp.float32),
    mesh=plsc.VectorSubcoreMesh(
        core_axis_name="c", subcore_axis_name="s", num_cores=NC,
        num_subcores=NS),
    compiler_params=pltpu.CompilerParams(
        needs_layout_passes=False, use_tc_tiling_on_sc=False,
        skip_device_barrier=True, disable_bounds_checks=True,
        disable_semaphore_checks=True),
    scratch_types=[
        pltpu.VMEM((TABLE_WORDS,), jnp.float32),
        pltpu.VMEM((RAYS_PER_W * 6,), jnp.float32),
        pltpu.VMEM((RAYS_PER_W,), jnp.int32),
        pltpu.VMEM((RAYS_PER_W,), jnp.int32),
        pltpu.VMEM((RAYS_PER_W * 6,), jnp.float32),
        pltpu.SemaphoreType.DMA,
        pltpu.SemaphoreType.DMA,
        pltpu.SemaphoreType.DMA,
        pltpu.SemaphoreType.DMA,
        pltpu.SemaphoreType.DMA,
    ],
)(_sc_body)


def kernel(correction_dict, rays, image_indices, depth_mask):
    table_t = correction_dict.astype(jnp.float32).T.reshape(-1)
    rays_t = rays.astype(jnp.float32).T.reshape(-1)
    out = _sc_kernel(table_t,
                     rays_t,
                     image_indices.reshape(-1).astype(jnp.int32),
                     depth_mask.reshape(-1).astype(jnp.int32))
    return out.reshape(6, N_RAYS).T


# speedup vs baseline: 1.0016x; 1.0016x over previous
import functools

import jax
import jax.numpy as jnp
from jax import lax
from jax.experimental import pallas as pl
from jax.experimental.pallas import tpu as pltpu
from jax.experimental.pallas import tpu_sc as plsc

N_FRAMES = 1000
N_RAYS = 16384
L = 16
NC = 1
NS = 16
NW = NC * NS
RAYS_PER_W = N_RAYS // NW
GROUPS = RAYS_PER_W // L
TABLE_WORDS = N_FRAMES * 7


def _sc_body(table_hbm, rays_hbm, idx_hbm, mask_hbm, out_hbm,
             table_v, rays_v, idx_v, mask_v, out_v,
             sem_t, sem_i, sem_m, sem_r, sem_o):
    wid = lax.axis_index("s") * NC + lax.axis_index("c")
    rbase = wid * RAYS_PER_W

    cp_t = pltpu.make_async_copy(table_hbm, table_v, sem_t)
    cp_i = pltpu.make_async_copy(idx_hbm.at[pl.ds(rbase, RAYS_PER_W)],
                                 idx_v, sem_i)
    cp_m = pltpu.make_async_copy(mask_hbm.at[pl.ds(rbase, RAYS_PER_W)],
                                 mask_v, sem_m)
    cp_r = pltpu.make_async_copy(
        rays_hbm.at[:, pl.ds(rbase, RAYS_PER_W)], rays_v, sem_r)
    cp_t.start(); cp_i.start(); cp_m.start(); cp_r.start()
    cp_t.wait(); cp_i.wait(); cp_m.wait(); cp_r.wait()

    zero = jnp.zeros((L,), jnp.float32)
    one = jnp.ones((L,), jnp.float32)

    @plsc.parallel_loop(0, GROUPS, unroll=1)
    def body(g):
        s = g * L
        idx = idx_v[pl.ds(s, L)]
        m = mask_v[pl.ds(s, L)] == 1
        c = [plsc.load_gather(table_v, [idx + k * N_FRAMES])
             for k in range(7)]
        tx = jnp.where(m, c[0], zero)
        ty = jnp.where(m, c[1], zero)
        tz = jnp.where(m, c[2], zero)
        qx = jnp.where(m, c[3], zero)
        qy = jnp.where(m, c[4], zero)
        qz = jnp.where(m, c[5], zero)
        qw = jnp.where(m, c[6], one)

        r = [rays_v[k, pl.ds(s, L)] for k in range(6)]

        xx, yy, zz = qx * qx, qy * qy, qz * qz
        xy, xz, yz = qx * qy, qx * qz, qy * qz
        wx, wy, wz = qw * qx, qw * qy, qw * qz
        two = jnp.float32(2.0)
        r00 = 1 - two * (yy + zz); r01 = two * (xy - wz); r02 = two * (xz + wy)
        r10 = two * (xy + wz); r11 = 1 - two * (xx + zz); r12 = two * (yz - wx)
        r20 = two * (xz - wy); r21 = two * (yz + wx); r22 = 1 - two * (xx + yy)

        out_v[0, pl.ds(s, L)] = r[0] + tx
        out_v[1, pl.ds(s, L)] = r[1] + ty
        out_v[2, pl.ds(s, L)] = r[2] + tz
        out_v[3, pl.ds(s, L)] = r00 * r[3] + r01 * r[4] + r02 * r[5]
        out_v[4, pl.ds(s, L)] = r10 * r[3] + r11 * r[4] + r12 * r[5]
        out_v[5, pl.ds(s, L)] = r20 * r[3] + r21 * r[4] + r22 * r[5]

    cp_o = pltpu.make_async_copy(
        out_v, out_hbm.at[:, pl.ds(rbase, RAYS_PER_W)], sem_o)
    cp_o.start()
    cp_o.wait()


_sc_kernel = functools.partial(
    pl.kernel,
    out_type=jax.ShapeDtypeStruct((6, N_RAYS), jnp.float32),
    mesh=plsc.VectorSubcoreMesh(
        core_axis_name="c", subcore_axis_name="s", num_cores=NC,
        num_subcores=NS),
    compiler_params=pltpu.CompilerParams(
        needs_layout_passes=False, use_tc_tiling_on_sc=False),
    scratch_types=[
        pltpu.VMEM((TABLE_WORDS,), jnp.float32),
        pltpu.VMEM((6, RAYS_PER_W), jnp.float32),
        pltpu.VMEM((RAYS_PER_W,), jnp.int32),
        pltpu.VMEM((RAYS_PER_W,), jnp.int32),
        pltpu.VMEM((6, RAYS_PER_W), jnp.float32),
        pltpu.SemaphoreType.DMA,
        pltpu.SemaphoreType.DMA,
        pltpu.SemaphoreType.DMA,
        pltpu.SemaphoreType.DMA,
        pltpu.SemaphoreType.DMA,
    ],
)(_sc_body)


def kernel(correction_dict, rays, image_indices, depth_mask):
    table_t = correction_dict.astype(jnp.float32).T.reshape(-1)
    rays_t = rays.astype(jnp.float32).T
    out = _sc_kernel(table_t,
                     rays_t,
                     image_indices.reshape(-1).astype(jnp.int32),
                     depth_mask.reshape(-1).astype(jnp.int32))
    return out.T
